# Initial kernel scaffold; baseline (speedup 1.0000x reference)
import jax
import jax.numpy as jnp
from jax.experimental import pallas as pl


def kernel(e_i, edge_index, edge_weight, W_g1, b_g1, W_a1, b_a1, W_a2, W_p1, b_p1, W_p2, b_p2):
    raise NotImplementedError("write your pallas kernel here")


# speedup vs baseline: 6.6480x; 6.6480x over previous
import functools

import jax
import jax.numpy as jnp
from jax import lax
from jax.experimental import pallas as pl
from jax.experimental.pallas import tpu as pltpu
from jax.experimental.pallas import tpu_sc as plsc

N = 10000
E = 160000
D = 384
H = 128
FH = 1

NC = 2
NS = 16
NW = NC * NS
C = 128
NCHUNK = E // C
ZR_BIG = 632
ZR_SMALL = 624


BLK1 = 2000


def _tc1_body(x_ref, w_ref, g_ref, q_ref):
    y = jnp.dot(x_ref[...], w_ref[...], preferred_element_type=jnp.float32)
    g_ref[...] = y[:, :H]
    q_ref[...] = y[:, H:]


def _tc1(e_i, wcat):
    return pl.pallas_call(
        _tc1_body,
        grid=(N // BLK1,),
        in_specs=[
            pl.BlockSpec((BLK1, D), lambda i: (i, 0)),
            pl.BlockSpec((D, 2 * H), lambda i: (0, 0)),
        ],
        out_specs=[
            pl.BlockSpec((BLK1, H), lambda i: (i, 0)),
            pl.BlockSpec((BLK1, H), lambda i: (i, 0)),
        ],
        out_shape=[
            jax.ShapeDtypeStruct((N, H), jnp.float32),
            jax.ShapeDtypeStruct((N, H), jnp.float32),
        ],
    )(e_i, wcat)


_GATHER_DNUMS = lax.GatherDimensionNumbers(
    offset_dims=(), collapsed_slice_dims=(0,), start_index_map=(0,))


def _sc_body(g_hbm, src_hbm, dst_hbm, w_hbm, z_hbm, out_hbm,
             srcv, dstv, wv, rows, acc, sem):
    cid = lax.axis_index("c")
    sid = lax.axis_index("s")
    wid = sid * NC + cid

    base_big = sid * ZR_BIG
    base_small = 2 * ZR_BIG + (sid - 2) * ZR_SMALL

    @pl.when(sid < 2)
    def _():
        pltpu.sync_copy(z_hbm, acc.at[pl.ds(pl.multiple_of(base_big, 8),
                                            ZR_BIG)])

    @pl.when(sid >= 2)
    def _():
        pltpu.sync_copy(z_hbm.at[pl.ds(0, ZR_SMALL)],
                        acc.at[pl.ds(pl.multiple_of(base_small, 8), ZR_SMALL)])

    plsc.subcore_barrier()

    nc_mine = jnp.where(wid < NCHUNK - (NCHUNK // NW) * NW,
                        NCHUNK // NW + 1, NCHUNK // NW)

    def chunk_body(k, carry):
        off = pl.multiple_of((wid + NW * k) * C, C)
        pltpu.sync_copy(src_hbm.at[pl.ds(off, C)], srcv)
        pltpu.sync_copy(dst_hbm.at[pl.ds(off, C)], dstv)
        pltpu.sync_copy(w_hbm.at[pl.ds(off, C)], wv)
        pltpu.async_copy(g_hbm.at[srcv], rows, sem).wait()

        def grp(i, c2):
            w16 = wv[pl.ds(i * 16, 16)]
            for j in range(16):
                wb = lax.gather(
                    w16, jnp.full((16, 1), j, jnp.int32), _GATHER_DNUMS, (1,),
                    mode=lax.GatherScatterMode.PROMISE_IN_BOUNDS)
                for kk in range(H // 16):
                    sl = rows[i * 16 + j, pl.ds(kk * 16, 16)]
                    rows[i * 16 + j, pl.ds(kk * 16, 16)] = sl * wb
            return c2

        lax.fori_loop(0, C // 16, grp, 0)
        pltpu.sync_copy(rows, acc.at[dstv], add=True)
        return carry

    lax.fori_loop(0, nc_mine, chunk_body, 0)
    plsc.subcore_barrier()

    @pl.when(sid < 2)
    def _():
        b = pl.multiple_of(base_big, 8)
        pltpu.sync_copy(acc.at[pl.ds(b, ZR_BIG)],
                        out_hbm.at[cid, pl.ds(b, ZR_BIG)])

    @pl.when(sid >= 2)
    def _():
        b = pl.multiple_of(base_small, 8)
        pltpu.sync_copy(acc.at[pl.ds(b, ZR_SMALL)],
                        out_hbm.at[cid, pl.ds(b, ZR_SMALL)])


@functools.cache
def _get_sc_segsum():
    mesh = plsc.VectorSubcoreMesh(core_axis_name="c", subcore_axis_name="s")
    return pl.kernel(
        _sc_body,
        mesh=mesh,
        out_type=jax.ShapeDtypeStruct((NC, N, H), jnp.float32),
        scratch_types=[
            pltpu.VMEM((C,), jnp.int32),
            pltpu.VMEM((C,), jnp.int32),
            pltpu.VMEM((C,), jnp.float32),
            pltpu.VMEM((C, H), jnp.float32),
            pltpu.VMEM_SHARED((N, H), jnp.float32),
            pltpu.SemaphoreType.DMA,
        ],
    )


BLK2 = 2000


def _tc2_body(a0_ref, a1_ref, q_ref, bg1_ref, wa1_ref, ba1_ref,
              wa2_ref, wp1_ref, bp1_ref, wp2_ref, bp2_ref, out_ref):
    h = jnp.maximum(a0_ref[...] + a1_ref[...] + bg1_ref[...], 0.0)
    z = jnp.maximum(
        jnp.dot(h, wa1_ref[...], preferred_element_type=jnp.float32)
        + ba1_ref[...], 0.0)
    wap = jnp.dot(wa2_ref[...], wp1_ref[...],
                  preferred_element_type=jnp.float32)
    p = jnp.maximum(
        q_ref[...] + jnp.dot(z, wap, preferred_element_type=jnp.float32)
        + bp1_ref[...], 0.0)
    out_ref[...] = (jnp.dot(p, wp2_ref[...], preferred_element_type=jnp.float32)
                    + bp2_ref[...])


def _tc2(a0, a1, q, b_g1, W_a1, b_a1, W_a2, W_p1, b_p1, W_p2, b_p2):
    row = lambda i: (i, 0)
    full = lambda i: (0, 0)
    return pl.pallas_call(
        _tc2_body,
        grid=(N // BLK2,),
        in_specs=[
            pl.BlockSpec((BLK2, H), row),
            pl.BlockSpec((BLK2, H), row),
            pl.BlockSpec((BLK2, H), row),
            pl.BlockSpec((1, H), full),
            pl.BlockSpec((H, H), full),
            pl.BlockSpec((1, H), full),
            pl.BlockSpec((H, D), full),
            pl.BlockSpec((D, H), full),
            pl.BlockSpec((1, H), full),
            pl.BlockSpec((H, FH), full),
            pl.BlockSpec((1, FH), full),
        ],
        out_specs=pl.BlockSpec((BLK2, FH), row),
        out_shape=jax.ShapeDtypeStruct((N, FH), jnp.float32),
    )(a0, a1, q, b_g1, W_a1, b_a1, W_a2, W_p1, b_p1, W_p2, b_p2)


def kernel(e_i, edge_index, edge_weight, W_g1, b_g1, W_a1, b_a1, W_a2,
           W_p1, b_p1, W_p2, b_p2):
    wcat = jnp.concatenate([W_g1, W_p1], axis=1)
    g, q = _tc1(e_i, wcat)

    zrows = jnp.zeros((ZR_BIG, H), jnp.float32)
    parts = _get_sc_segsum()(g, edge_index[0], edge_index[1], edge_weight,
                             zrows)

    return _tc2(parts[0], parts[1], q,
                b_g1.reshape(1, H), W_a1, b_a1.reshape(1, H), W_a2,
                W_p1, b_p1.reshape(1, H), W_p2, b_p2.reshape(1, FH))
